# baseline (speedup 1.0000x reference)
import functools

import jax
import jax.numpy as jnp
from jax import lax
from jax.experimental import pallas as pl
from jax.experimental.pallas import tpu as pltpu
from jax.experimental.pallas import tpu_sc as plsc

EMBED_DIM = 32
LANES = 16
GCOLS = 512


@functools.cache
def _build_transpose(V, D):
    gt = -(-V // GCOLS)
    nq = gt * 128

    def body(in_ref, out_ref):
        blk = in_ref[...]
        parts = [jnp.transpose(blk[:, u * 128:(u + 1) * 128]) for u in range(4)]
        out_ref[...] = jnp.concatenate(parts, axis=1)

    return pl.pallas_call(
        body,
        grid=(gt,),
        in_specs=[pl.BlockSpec((D, GCOLS), lambda t: (0, t))],
        out_specs=pl.BlockSpec((128, 128), lambda t: (t, 0)),
        out_shape=jax.ShapeDtypeStruct((nq, 128), jnp.float32),
    )


@functools.cache
def _build_lookup(NI, NJ, NQ):
    info = plsc.get_sparse_core_info()
    NC = info.num_cores
    NW = NC * info.num_subcores
    IB = NI // NW
    assert IB == 128 and NJ % 2 == 0

    mesh = plsc.VectorSubcoreMesh(core_axis_name="c", subcore_axis_name="s")

    @functools.partial(
        pl.kernel,
        mesh=mesh,
        compiler_params=pltpu.CompilerParams(
            use_tc_tiling_on_sc=False, needs_layout_passes=False
        ),
        out_type=jax.ShapeDtypeStruct((NJ, EMBED_DIM, NI), jnp.float32),
        scratch_types=(
            [
                pltpu.VMEM((NJ, IB), jnp.int32),
                pltpu.VMEM((NJ, IB), jnp.int32),
                pltpu.VMEM((2, IB, 128), jnp.float32),
                pltpu.VMEM((2, EMBED_DIM, IB), jnp.float32),
            ]
            + [pltpu.SemaphoreType.DMA] * 4
        ),
    )
    def emb_kernel(xt_hbm, tab_hbm, out_hbm, gidx, offv, gbuf, obuf,
                   gsem0, gsem1, osem0, osem1):
        gsem = (gsem0, gsem1)
        osem = (osem0, osem1)
        wid = lax.axis_index("s") * NC + lax.axis_index("c")
        ibase = wid * IB

        pltpu.sync_copy(xt_hbm.at[:, pl.ds(ibase, IB)], gidx)

        def prep(t, carry):
            for k in range(IB // LANES):
                s = pl.ds(k * LANES, LANES)
                v = gidx[t, s]
                offv[t, s] = ((v >> 7) & 3) * 32
                gidx[t, s] = ((v >> 9) * 128) + (v & 127)
            return carry

        lax.fori_loop(0, NJ, prep, 0)

        def gather_start(j, b):
            pltpu.make_async_copy(
                tab_hbm.at[gidx.at[j]], gbuf.at[b], gsem[b]
            ).start()

        def gather_wait(b):
            pltpu.make_async_copy(
                tab_hbm.at[gidx.at[0]], gbuf.at[b], gsem[b]
            ).wait()

        def out_start(j, b):
            pltpu.make_async_copy(
                obuf.at[b], out_hbm.at[j, :, pl.ds(ibase, IB)], osem[b]
            ).start()

        def out_wait(b):
            pltpu.make_async_copy(
                obuf.at[b], out_hbm.at[0, :, pl.ds(ibase, IB)], osem[b]
            ).wait()

        def extract(j, b):
            for k in range(IB // LANES):
                rows = lax.iota(jnp.int32, LANES) + (k * LANES)
                cols0 = offv[j, pl.ds(k * LANES, LANES)]
                for c in range(EMBED_DIM):
                    obuf[b, c, pl.ds(k * LANES, LANES)] = plsc.load_gather(
                        gbuf.at[b], [rows, cols0 + c]
                    )

        gather_start(0, 0)
        gather_start(1, 1)

        def step(o, carry):
            for b in range(2):
                j = o * 2 + b
                gather_wait(b)
                pl.when(j >= 2)(lambda b=b: out_wait(b))
                extract(j, b)
                out_start(j, b)
                pl.when(j + 2 < NJ)(lambda j=j, b=b: gather_start(j + 2, b))
            return carry

        lax.fori_loop(0, NJ // 2, step, 0)
        out_wait(0)
        out_wait(1)

    return emb_kernel


def kernel(x, table):
    NI, NJ = x.shape
    V, D = table.shape
    tab4 = _build_transpose(V, D)(table.T)
    fn = _build_lookup(NI, NJ, tab4.shape[0])
    xt = x.T.astype(jnp.int32)
    out_t = fn(xt, tab4)
    return out_t.transpose(2, 0, 1)


# speedup vs baseline: 2.9478x; 2.9478x over previous
import functools

import jax
import jax.numpy as jnp
from jax import lax
from jax.experimental import pallas as pl
from jax.experimental.pallas import tpu as pltpu
from jax.experimental.pallas import tpu_sc as plsc

EMBED_DIM = 32
LANES = 16
GCOLS = 4096


@functools.cache
def _build_transpose(V, D):
    gt = -(-V // GCOLS)
    nq = gt * (GCOLS // 4)

    def body(in_ref, out_ref):
        eye = (lax.broadcasted_iota(jnp.int32, (128, 128), 0)
               == lax.broadcasted_iota(jnp.int32, (128, 128), 1)
               ).astype(jnp.float32)
        for g in range(GCOLS // 512):
            blk = in_ref[:, pl.ds(g * 512, 512)]
            parts = [
                lax.dot_general(
                    eye, blk[:, u * 128:(u + 1) * 128],
                    ((( 1,), (1,)), ((), ())),
                    preferred_element_type=jnp.float32,
                )
                for u in range(4)
            ]
            out_ref[pl.ds(g * 128, 128), :] = jnp.concatenate(parts, axis=1)

    return pl.pallas_call(
        body,
        grid=(gt,),
        in_specs=[pl.BlockSpec((D, GCOLS), lambda t: (0, t))],
        out_specs=pl.BlockSpec((GCOLS // 4, 128), lambda t: (t, 0)),
        out_shape=jax.ShapeDtypeStruct((nq, 128), jnp.float32),
    )


@functools.cache
def _build_lookup(NI, NJ, NQ):
    info = plsc.get_sparse_core_info()
    NC = info.num_cores
    NW = NC * info.num_subcores
    IB = NI // NW
    assert IB == 128 and NJ % 2 == 0

    mesh = plsc.VectorSubcoreMesh(core_axis_name="c", subcore_axis_name="s")

    @functools.partial(
        pl.kernel,
        mesh=mesh,
        compiler_params=pltpu.CompilerParams(
            use_tc_tiling_on_sc=False, needs_layout_passes=False
        ),
        out_type=jax.ShapeDtypeStruct((NJ, EMBED_DIM, NI), jnp.float32),
        scratch_types=(
            [
                pltpu.VMEM((NJ, IB), jnp.int32),
                pltpu.VMEM((NJ, IB), jnp.int32),
                pltpu.VMEM((2, IB, 128), jnp.float32),
                pltpu.VMEM((2, EMBED_DIM, IB), jnp.float32),
            ]
            + [pltpu.SemaphoreType.DMA] * 4
        ),
    )
    def emb_kernel(xt_hbm, tab_hbm, out_hbm, gidx, offv, gbuf, obuf,
                   gsem0, gsem1, osem0, osem1):
        gsem = (gsem0, gsem1)
        osem = (osem0, osem1)
        wid = lax.axis_index("s") * NC + lax.axis_index("c")
        ibase = wid * IB

        pltpu.sync_copy(xt_hbm.at[:, pl.ds(ibase, IB)], gidx)

        def prep(t, carry):
            for k in range(IB // LANES):
                s = pl.ds(k * LANES, LANES)
                v = gidx[t, s]
                offv[t, s] = ((v >> 7) & 3) * 32
                gidx[t, s] = ((v >> 9) * 128) + (v & 127)
            return carry

        lax.fori_loop(0, NJ, prep, 0)

        def gather_start(j, b):
            pltpu.make_async_copy(
                tab_hbm.at[gidx.at[j]], gbuf.at[b], gsem[b]
            ).start()

        def gather_wait(b):
            pltpu.make_async_copy(
                tab_hbm.at[gidx.at[0]], gbuf.at[b], gsem[b]
            ).wait()

        def out_start(j, b):
            pltpu.make_async_copy(
                obuf.at[b], out_hbm.at[j, :, pl.ds(ibase, IB)], osem[b]
            ).start()

        def out_wait(b):
            pltpu.make_async_copy(
                obuf.at[b], out_hbm.at[0, :, pl.ds(ibase, IB)], osem[b]
            ).wait()

        def extract(j, b):
            for k in range(IB // LANES):
                rows = lax.iota(jnp.int32, LANES) + (k * LANES)
                cols0 = offv[j, pl.ds(k * LANES, LANES)]
                for c in range(EMBED_DIM):
                    obuf[b, c, pl.ds(k * LANES, LANES)] = plsc.load_gather(
                        gbuf.at[b], [rows, cols0 + c]
                    )

        gather_start(0, 0)
        gather_start(1, 1)

        def step(o, carry):
            for b in range(2):
                j = o * 2 + b
                gather_wait(b)
                pl.when(j >= 2)(lambda b=b: out_wait(b))
                extract(j, b)
                out_start(j, b)
                pl.when(j + 2 < NJ)(lambda j=j, b=b: gather_start(j + 2, b))
            return carry

        lax.fori_loop(0, NJ // 2, step, 0)
        out_wait(0)
        out_wait(1)

    return emb_kernel


def kernel(x, table):
    NI, NJ = x.shape
    V, D = table.shape
    tab4 = _build_transpose(V, D)(table.T)
    fn = _build_lookup(NI, NJ, tab4.shape[0])
    xt = x.T.astype(jnp.int32)
    out_t = fn(xt, tab4)
    return out_t.transpose(2, 0, 1)
